# baseline (speedup 1.0000x reference)
import functools

import jax
import jax.numpy as jnp
from jax import lax
from jax.experimental import pallas as pl
from jax.experimental.pallas import tpu as pltpu
from jax.experimental.pallas import tpu_sc as plsc

N_NODES = 10000
N_EDGES = 320000
D_BLOCK = 128
D_EDGE = 16
D_OUT = D_EDGE + 2 * D_BLOCK
LANES = 16

NC = 2
NS = 16
NW = NC * NS

CHUNK = 128
HALF = CHUNK // 2
N_CHUNKS_TOT = N_EDGES // CHUNK
STEPS = -(-N_CHUNKS_TOT // NW)
N_PAIRS = (STEPS - 1) // 2


def _make_router():
    mesh = plsc.VectorSubcoreMesh(core_axis_name="c", subcore_axis_name="s")

    idx_t = pltpu.VMEM((HALF,), jnp.int32)
    per_set = dict(
        sidx_lo=idx_t, sidx_hi=idx_t, didx_lo=idx_t, didx_hi=idx_t,
        obufS=pltpu.VMEM((D_EDGE + D_BLOCK, CHUNK), jnp.float32),
        obufD=pltpu.VMEM((D_BLOCK, CHUNK), jnp.float32),
    )

    @functools.partial(
        pl.kernel,
        out_type=jax.ShapeDtypeStruct((D_OUT, N_EDGES), jnp.float32),
        mesh=mesh,
        compiler_params=pltpu.CompilerParams(needs_layout_passes=False),
        scratch_types=(
            [v for v in per_set.values()] * 2
            + [pltpu.VMEM((HALF, D_BLOCK), jnp.float32)] * 2
            + [pltpu.SemaphoreType.DMA] * 10
        ),
    )
    def router(tbl_hbm, attrT_hbm, eidx_hbm, outT_hbm,
               slA, shA, dlA, dhA, obufSA, obufDA,
               slB, shB, dlB, dhB, obufSB, obufDB,
               rows0, rows1,
               isemA, isemB, asemA, asemB, osemSA, osemSB, osemDA, osemDB,
               gsem0, gsem1):
        wid = lax.axis_index("s") * NC + lax.axis_index("c")

        A = dict(sl=slA, sh=shA, dl=dlA, dh=dhA, obufS=obufSA, obufD=obufDA,
                 isem=isemA, asem=asemA, osemS=osemSA, osemD=osemDA)
        B = dict(sl=slB, sh=shB, dl=dlB, dh=dhB, obufS=obufSB, obufD=obufDB,
                 isem=isemB, asem=asemB, osemS=osemSB, osemD=osemDB)

        def eoff(step):
            g = wid + NW * step
            g = jnp.where(g < N_CHUNKS_TOT, g, g - NW)
            return g * CHUNK

        def issue_idx(step, s):
            es = eoff(step)
            pltpu.async_copy(eidx_hbm.at[pl.ds(es, HALF)], s["sl"], s["isem"])
            pltpu.async_copy(eidx_hbm.at[pl.ds(es + HALF, HALF)], s["sh"], s["isem"])
            pltpu.async_copy(
                eidx_hbm.at[pl.ds(N_EDGES + es, HALF)], s["dl"], s["isem"])
            pltpu.async_copy(
                eidx_hbm.at[pl.ds(N_EDGES + es + HALF, HALF)], s["dh"], s["isem"])

        def wait_idx(s):
            for _ in range(4):
                pltpu.make_async_copy(
                    eidx_hbm.at[pl.ds(0, HALF)], s["sl"], s["isem"]).wait()

        def gather(idx_buf, rows, gsem):
            pltpu.async_copy(tbl_hbm.at[idx_buf], rows, gsem)

        def wait_gather(rows, gsem):
            pltpu.make_async_copy(tbl_hbm.at[slA], rows, gsem).wait()

        e_iotas = [lax.iota(jnp.int32, 16) + g * LANES for g in range(4)]

        def fill_half(rows, obuf, band, col0):
            def jgroup(jj, c):
                for u in range(4):
                    j = jj * 4 + u
                    jvec = jnp.full((16,), j, jnp.int32)
                    for g in range(4):
                        vals = plsc.load_gather(rows, [e_iotas[g], jvec])
                        obuf[band + j, pl.ds(col0 + g * LANES, LANES)] = vals
                return c

            lax.fori_loop(0, D_BLOCK // 4, jgroup, 0)

        def store_S(es, s):
            pltpu.async_copy(
                s["obufS"],
                outT_hbm.at[pl.ds(0, D_EDGE + D_BLOCK), pl.ds(es, CHUNK)],
                s["osemS"])

        def wait_store_S(s):
            pltpu.make_async_copy(
                s["obufS"],
                outT_hbm.at[pl.ds(0, D_EDGE + D_BLOCK), pl.ds(0, CHUNK)],
                s["osemS"]).wait()

        def store_D(es, s):
            pltpu.async_copy(
                s["obufD"],
                outT_hbm.at[pl.ds(D_EDGE + D_BLOCK, D_BLOCK), pl.ds(es, CHUNK)],
                s["osemD"])

        def wait_store_D(s):
            pltpu.make_async_copy(
                s["obufD"],
                outT_hbm.at[pl.ds(D_EDGE + D_BLOCK, D_BLOCK), pl.ds(0, CHUNK)],
                s["osemD"]).wait()

        def chunk_body(step, s, s_next, guarded):
            es = eoff(step)

            def waits_S():
                wait_store_S(s)
                wait_store_D(s)

            if guarded is True:
                waits_S()
            else:
                pl.when(guarded)(waits_S)

            pltpu.async_copy(
                attrT_hbm.at[:, pl.ds(es, CHUNK)],
                s["obufS"].at[pl.ds(0, D_EDGE), :], s["asem"])

            wait_gather(rows0, gsem0)
            fill_half(rows0, s["obufS"], D_EDGE, 0)
            gather(s["dl"], rows0, gsem0)
            wait_gather(rows1, gsem1)
            fill_half(rows1, s["obufS"], D_EDGE, HALF)
            gather(s["dh"], rows1, gsem1)
            pltpu.make_async_copy(
                attrT_hbm.at[:, pl.ds(0, CHUNK)],
                s["obufS"].at[pl.ds(0, D_EDGE), :], s["asem"]).wait()
            store_S(es, s)

            wait_gather(rows0, gsem0)
            fill_half(rows0, s["obufD"], 0, 0)
            wait_idx(s_next)
            gather(s_next["sl"], rows0, gsem0)
            wait_gather(rows1, gsem1)
            fill_half(rows1, s["obufD"], 0, HALF)
            gather(s_next["sh"], rows1, gsem1)
            store_D(es, s)

            issue_idx(step + 2, s)

        issue_idx(0, A)
        wait_idx(A)
        gather(A["sl"], rows0, gsem0)
        gather(A["sh"], rows1, gsem1)
        issue_idx(1, B)

        def body(k, carry):
            chunk_body(2 * k, A, B, k > 0)
            chunk_body(2 * k + 1, B, A, k > 0)
            return carry

        lax.fori_loop(0, N_PAIRS, body, 0)

        chunk_body(STEPS - 1, A, B, True)
        wait_gather(rows0, gsem0)
        wait_gather(rows1, gsem1)
        wait_idx(A)
        wait_store_S(A)
        wait_store_D(A)
        wait_store_S(B)
        wait_store_D(B)

    return router


_router = _make_router()


def kernel(block_input, raw_input, edge_attr, edge_index):
    del raw_input
    eidx_flat = edge_index.astype(jnp.int32).reshape(-1)
    outT = _router(block_input, edge_attr.T, eidx_flat)
    return outT.T


# speedup vs baseline: 3.0168x; 3.0168x over previous
import functools

import jax
import jax.numpy as jnp
from jax import lax
from jax.experimental import pallas as pl
from jax.experimental.pallas import tpu as pltpu
from jax.experimental.pallas import tpu_sc as plsc

N_NODES = 10000
N_EDGES = 320000
D_BLOCK = 128
D_EDGE = 16
D_OUT = D_EDGE + 2 * D_BLOCK
LANES = 16

NC = 2
NS = 16
NW = NC * NS

CHUNK = 128
HALF = CHUNK // 2
N_CHUNKS_TOT = N_EDGES // CHUNK
STEPS = -(-N_CHUNKS_TOT // NW)
N_PAIRS = (STEPS - 1) // 2


def _make_router():
    mesh = plsc.VectorSubcoreMesh(core_axis_name="c", subcore_axis_name="s")

    idx_t = pltpu.VMEM((HALF,), jnp.int32)
    per_set = dict(
        sidx_lo=idx_t, sidx_hi=idx_t, didx_lo=idx_t, didx_hi=idx_t,
        obufS=pltpu.VMEM((D_EDGE + D_BLOCK, CHUNK), jnp.float32),
        obufD=pltpu.VMEM((D_BLOCK, CHUNK), jnp.float32),
    )

    @functools.partial(
        pl.kernel,
        out_type=jax.ShapeDtypeStruct((D_OUT, N_EDGES), jnp.float32),
        mesh=mesh,
        compiler_params=pltpu.CompilerParams(needs_layout_passes=False),
        scratch_types=(
            [v for v in per_set.values()] * 2
            + [pltpu.VMEM((HALF, D_BLOCK), jnp.float32)] * 2
            + [pltpu.SemaphoreType.DMA] * 10
        ),
    )
    def router(tbl_hbm, attrT_hbm, eidx_hbm, outT_hbm,
               slA, shA, dlA, dhA, obufSA, obufDA,
               slB, shB, dlB, dhB, obufSB, obufDB,
               rows0, rows1,
               isemA, isemB, asemA, asemB, osemSA, osemSB, osemDA, osemDB,
               gsem0, gsem1):
        wid = lax.axis_index("s") * NC + lax.axis_index("c")

        A = dict(sl=slA, sh=shA, dl=dlA, dh=dhA, obufS=obufSA, obufD=obufDA,
                 isem=isemA, asem=asemA, osemS=osemSA, osemD=osemDA)
        B = dict(sl=slB, sh=shB, dl=dlB, dh=dhB, obufS=obufSB, obufD=obufDB,
                 isem=isemB, asem=asemB, osemS=osemSB, osemD=osemDB)

        def eoff(step):
            g = wid + NW * step
            g = jnp.where(g < N_CHUNKS_TOT, g, g - NW)
            return g * CHUNK

        def issue_idx(step, s):
            es = eoff(step)
            pltpu.async_copy(eidx_hbm.at[pl.ds(es, HALF)], s["sl"], s["isem"])
            pltpu.async_copy(eidx_hbm.at[pl.ds(es + HALF, HALF)], s["sh"], s["isem"])
            pltpu.async_copy(
                eidx_hbm.at[pl.ds(N_EDGES + es, HALF)], s["dl"], s["isem"])
            pltpu.async_copy(
                eidx_hbm.at[pl.ds(N_EDGES + es + HALF, HALF)], s["dh"], s["isem"])

        def wait_idx(s):
            for _ in range(4):
                pltpu.make_async_copy(
                    eidx_hbm.at[pl.ds(0, HALF)], s["sl"], s["isem"]).wait()

        def gather(idx_buf, rows, gsem):
            pltpu.async_copy(tbl_hbm.at[idx_buf], rows, gsem)

        def wait_gather(rows, gsem):
            pltpu.make_async_copy(tbl_hbm.at[slA], rows, gsem).wait()

        iota16 = lax.iota(jnp.int32, LANES)
        rots = [(iota16 + d) & (LANES - 1) for d in range(LANES)]

        def fill_half(rows, obuf, band, col0):
            bandvec = jnp.full((LANES,), band, jnp.int32)

            def hblock(h, c):
                h16 = h * LANES
                lcols = [jnp.full((LANES,), h16, jnp.int32) + r for r in rots]

                def gblock(g, c2):
                    g16 = g * LANES
                    egv = jnp.full((LANES,), g16, jnp.int32) + iota16
                    scg = jnp.full((LANES,), col0 + g16, jnp.int32) + iota16
                    for d in range(LANES):
                        vals = plsc.load_gather(rows, [egv, lcols[d]])
                        plsc.store_scatter(
                            obuf, [lcols[d] + bandvec, scg], vals)
                    return c2

                lax.fori_loop(0, HALF // LANES, gblock, 0)
                return c

            lax.fori_loop(0, D_BLOCK // LANES, hblock, 0)

        def store_S(es, s):
            pltpu.async_copy(
                s["obufS"],
                outT_hbm.at[pl.ds(0, D_EDGE + D_BLOCK), pl.ds(es, CHUNK)],
                s["osemS"])

        def wait_store_S(s):
            pltpu.make_async_copy(
                s["obufS"],
                outT_hbm.at[pl.ds(0, D_EDGE + D_BLOCK), pl.ds(0, CHUNK)],
                s["osemS"]).wait()

        def store_D(es, s):
            pltpu.async_copy(
                s["obufD"],
                outT_hbm.at[pl.ds(D_EDGE + D_BLOCK, D_BLOCK), pl.ds(es, CHUNK)],
                s["osemD"])

        def wait_store_D(s):
            pltpu.make_async_copy(
                s["obufD"],
                outT_hbm.at[pl.ds(D_EDGE + D_BLOCK, D_BLOCK), pl.ds(0, CHUNK)],
                s["osemD"]).wait()

        def chunk_body(step, s, s_next, guarded):
            es = eoff(step)

            def waits_S():
                wait_store_S(s)
                wait_store_D(s)

            if guarded is True:
                waits_S()
            else:
                pl.when(guarded)(waits_S)

            pltpu.async_copy(
                attrT_hbm.at[:, pl.ds(es, CHUNK)],
                s["obufS"].at[pl.ds(0, D_EDGE), :], s["asem"])

            wait_gather(rows0, gsem0)
            fill_half(rows0, s["obufS"], D_EDGE, 0)
            gather(s["dl"], rows0, gsem0)
            wait_gather(rows1, gsem1)
            fill_half(rows1, s["obufS"], D_EDGE, HALF)
            gather(s["dh"], rows1, gsem1)
            pltpu.make_async_copy(
                attrT_hbm.at[:, pl.ds(0, CHUNK)],
                s["obufS"].at[pl.ds(0, D_EDGE), :], s["asem"]).wait()
            store_S(es, s)

            wait_gather(rows0, gsem0)
            fill_half(rows0, s["obufD"], 0, 0)
            wait_idx(s_next)
            gather(s_next["sl"], rows0, gsem0)
            wait_gather(rows1, gsem1)
            fill_half(rows1, s["obufD"], 0, HALF)
            gather(s_next["sh"], rows1, gsem1)
            store_D(es, s)

            issue_idx(step + 2, s)

        issue_idx(0, A)
        wait_idx(A)
        gather(A["sl"], rows0, gsem0)
        gather(A["sh"], rows1, gsem1)
        issue_idx(1, B)

        def body(k, carry):
            chunk_body(2 * k, A, B, k > 0)

            @pl.when(k < N_PAIRS)
            def _():
                chunk_body(2 * k + 1, B, A, k > 0)

            return carry

        lax.fori_loop(0, N_PAIRS + 1, body, 0)

        wait_gather(rows0, gsem0)
        wait_gather(rows1, gsem1)
        wait_idx(A)
        wait_store_S(A)
        wait_store_D(A)
        wait_store_S(B)
        wait_store_D(B)

    return router


_router = _make_router()


def kernel(block_input, raw_input, edge_attr, edge_index):
    del raw_input
    eidx_flat = edge_index.astype(jnp.int32).reshape(-1)
    outT = _router(block_input, edge_attr.T, eidx_flat)
    return outT.T
